# baseline (speedup 1.0000x reference)
import functools

import jax
import jax.numpy as jnp
from jax import lax
from jax.experimental import pallas as pl
from jax.experimental.pallas import tpu as pltpu
from jax.experimental.pallas import tpu_sc as plsc

_RADII = [[0.1, 0.5], [0.5, 1.0], [1.0, 2.0], [2.0, 4.0]]
_NPOINTS = [512, 256, 128, 64]
_NSAMPLES = [16, 32]


_NC, _NS = 2, 16
_NW = _NC * _NS
_CHUNK_BYTES = 262144


def _chunk_rows(b_per_w, c):
    cap = max(8, (_CHUNK_BYTES // (4 * c)) // 8 * 8)
    ch = 8
    for d in range(8, cap + 1, 8):
        if b_per_w % d == 0:
            ch = d
    return ch


@functools.partial(jax.jit, static_argnums=(2,))
def _sc_gather_rows(table, idx, c):
    m = idx.shape[0]
    b_per_w = m // _NW
    ch = _chunk_rows(b_per_w, c)
    n_chunks = b_per_w // ch
    mesh = plsc.VectorSubcoreMesh(core_axis_name="c", subcore_axis_name="s")

    @functools.partial(
        pl.kernel, mesh=mesh,
        compiler_params=pltpu.CompilerParams(use_tc_tiling_on_sc=False),
        out_type=jax.ShapeDtypeStruct((m, c), jnp.float32),
        scratch_types=[
            pltpu.VMEM((ch,), jnp.int32),
            pltpu.VMEM((ch, c), jnp.float32),
            pltpu.SemaphoreType.DMA,
        ],
    )
    def k(table_hbm, idx_hbm, out_hbm, idx_v, rows_v, sem):
        wid = lax.axis_index("s") * _NC + lax.axis_index("c")
        base = wid * b_per_w

        def body(t, _):
            off = base + t * ch
            pltpu.sync_copy(idx_hbm.at[pl.ds(off, ch)], idx_v)
            pltpu.async_copy(table_hbm.at[idx_v], rows_v, sem).wait()
            pltpu.sync_copy(rows_v, out_hbm.at[pl.ds(off, ch)])
            return 0

        lax.fori_loop(0, n_chunks, body, 0)

    return k(table, idx)


def _pad_lanes(x):
    c = x.shape[-1]
    cp = -(-c // 16) * 16
    if cp == c:
        return x
    pad = [(0, 0)] * (x.ndim - 1) + [(0, cp - c)]
    return jnp.pad(x, pad)


def _index_points(points, idx):
    b, n, c = points.shape
    table = _pad_lanes(points).reshape(b * n, -1)
    idx = jnp.clip(idx, 0, n - 1)
    flat = (idx + (jnp.arange(b, dtype=jnp.int32) * n).reshape(
        (b,) + (1,) * (idx.ndim - 1))).reshape(-1).astype(jnp.int32)
    rows = _sc_gather_rows(table, flat, table.shape[-1])
    out = rows[:, :c].reshape(idx.shape + (c,))
    return lax.optimization_barrier(out)


def _square_distance(src, dst):
    d = -2.0 * jnp.einsum('bsc,bnc->bsn', src, dst)
    d = d + jnp.sum(src ** 2, axis=-1)[:, :, None]
    d = d + jnp.sum(dst ** 2, axis=-1)[:, None, :]
    return d


def _fps(xyz, npoint):
    b, n, _ = xyz.shape
    def body(i, state):
        centroids, distance, farthest = state
        centroids = centroids.at[:, i].set(farthest)
        centroid = jnp.take_along_axis(
            xyz, jnp.broadcast_to(farthest[:, None, None], (b, 1, 3)), axis=1)
        dist = jnp.sum((xyz - centroid) ** 2, axis=-1)
        distance = jnp.minimum(distance, dist)
        farthest = jnp.argmax(distance, axis=-1).astype(jnp.int32)
        return centroids, distance, farthest
    init = (jnp.zeros((b, npoint), jnp.int32),
            jnp.full((b, n), 1e10, jnp.float32), jnp.zeros((b,), jnp.int32))
    centroids, _, _ = jax.lax.fori_loop(0, npoint, body, init)
    return centroids


def _ball_query(radius, nsample, xyz, new_xyz):
    b, n, _ = xyz.shape
    s = new_xyz.shape[1]
    sqrdists = _square_distance(new_xyz, xyz)
    group_idx = jnp.broadcast_to(jnp.arange(n, dtype=jnp.int32), (b, s, n))
    group_idx = jnp.where(sqrdists > radius ** 2, n, group_idx)
    group_idx = jnp.sort(group_idx, axis=-1)[:, :, :nsample]
    group_first = group_idx[:, :, :1]
    group_idx = jnp.where(group_idx == n, group_first, group_idx)
    return group_idx


def _conv_bn_relu(x, W, b, g, be):
    y = jnp.einsum('oc,bcsk->bosk', W, x) + b[None, :, None, None]
    mean = jnp.mean(y, axis=(0, 2, 3), keepdims=True)
    var = jnp.mean((y - mean) ** 2, axis=(0, 2, 3), keepdims=True)
    y = (y - mean) / jnp.sqrt(var + 1e-5)
    y = y * g[None, :, None, None] + be[None, :, None, None]
    return jax.nn.relu(y)


def _sa_module(xyz, features, npoint, radii, nsamples, scale_params):
    b, n, _ = xyz.shape
    fps_idx = _fps(xyz, npoint)
    new_xyz = _index_points(xyz, fps_idx)
    feats_t = jnp.transpose(features, (0, 2, 1))
    cat = jnp.concatenate([xyz, feats_t], axis=-1)
    cf = cat.shape[-1]
    table = _pad_lanes(cat).reshape(b * n, -1)

    idxs = [_ball_query(radius, nsample, xyz, new_xyz)
            for radius, nsample in zip(radii, nsamples)]
    all_idx = jnp.concatenate(idxs, axis=-1)
    all_idx = jnp.clip(all_idx, 0, n - 1)
    flat = (all_idx + (jnp.arange(b, dtype=jnp.int32) * n)[:, None, None]
            ).reshape(-1).astype(jnp.int32)
    rows = _sc_gather_rows(table, flat, table.shape[-1])
    rows = rows.reshape(all_idx.shape + (table.shape[-1],))

    outs = []
    k_off = 0
    for (radius, nsample, layers) in zip(radii, nsamples, scale_params):
        part = rows[:, :, k_off:k_off + nsample, :]
        k_off += nsample
        grouped_xyz = lax.optimization_barrier(
            part[..., :3]) - new_xyz[:, :, None, :]
        nf = jnp.transpose(grouped_xyz, (0, 3, 1, 2))
        gf = jnp.transpose(
            lax.optimization_barrier(part[..., 3:cf]), (0, 3, 1, 2))
        nf = jnp.concatenate([nf, gf], axis=1)
        for (W, b_, g, be) in layers:
            nf = _conv_bn_relu(nf, W, b_, g, be)
        outs.append(jnp.max(nf, axis=-1))
    return new_xyz, jnp.concatenate(outs, axis=1)


def _fp_module(unknown, known, unknow_feats, known_feats, layers):
    d = _square_distance(unknown, known)
    neg_dist, idx = jax.lax.top_k(-d, 3)
    dist = jnp.maximum(-neg_dist, 0.0)
    dist_recip = 1.0 / (dist + 1e-8)
    norm = jnp.sum(dist_recip, axis=2, keepdims=True)
    weight = dist_recip / norm
    neighbors = _index_points(jnp.transpose(known_feats, (0, 2, 1)), idx)
    interpolated = jnp.transpose(
        jnp.sum(neighbors * weight[..., None], axis=2), (0, 2, 1))
    if unknow_feats is not None:
        x = jnp.concatenate([interpolated, unknow_feats], axis=1)
    else:
        x = interpolated
    x = x[..., None]
    for (W, b, g, be) in layers:
        x = _conv_bn_relu(x, W, b, g, be)
    return x[..., 0]


def _identity_pallas(x):
    def body(x_ref, o_ref):
        o_ref[...] = x_ref[...]
    b = x.shape[0]
    blk = (1,) + x.shape[1:]
    idx = lambda i: (i,) + (0,) * (len(x.shape) - 1)
    return pl.pallas_call(
        body,
        grid=(b,),
        in_specs=[pl.BlockSpec(blk, idx)],
        out_specs=pl.BlockSpec(blk, idx),
        out_shape=jax.ShapeDtypeStruct(x.shape, x.dtype))(x)


def kernel(pointcloud, params):
    xyz = pointcloud[..., 0:3]
    features = jnp.transpose(pointcloud[..., 3:], (0, 2, 1))
    l_xyz = [xyz]
    l_features = [features]
    for i in range(4):
        nx, nf = _sa_module(l_xyz[i], l_features[i], _NPOINTS[i], _RADII[i],
                            _NSAMPLES, params['sa'][i])
        l_xyz.append(nx)
        l_features.append(nf)
    for i in range(-1, -5, -1):
        l_features[i - 1] = _fp_module(
            l_xyz[i - 1], l_xyz[i], l_features[i - 1], l_features[i],
            params['fp'][i])
    return _identity_pallas(l_features[0])


# speedup vs baseline: 1.0145x; 1.0145x over previous
import functools

import jax
import jax.numpy as jnp
from jax import lax
from jax.experimental import pallas as pl
from jax.experimental.pallas import tpu as pltpu
from jax.experimental.pallas import tpu_sc as plsc

_RADII = [[0.1, 0.5], [0.5, 1.0], [1.0, 2.0], [2.0, 4.0]]
_NPOINTS = [512, 256, 128, 64]
_NSAMPLES = [16, 32]


_NC, _NS = 2, 16
_NW = _NC * _NS
_CHUNK_BYTES = 262144


def _chunk_rows(b_per_w, c):
    cap = max(8, (_CHUNK_BYTES // (4 * c)) // 8 * 8)
    ch = 8
    for d in range(8, cap + 1, 8):
        if b_per_w % d == 0:
            ch = d
    return ch


@functools.partial(jax.jit, static_argnums=(2,))
def _sc_gather_rows(table, idx, c):
    m = idx.shape[0]
    b_per_w = m // _NW
    ch = _chunk_rows(b_per_w, c)
    n_chunks = b_per_w // ch
    mesh = plsc.VectorSubcoreMesh(core_axis_name="c", subcore_axis_name="s")

    @functools.partial(
        pl.kernel, mesh=mesh,
        compiler_params=pltpu.CompilerParams(use_tc_tiling_on_sc=False),
        out_type=jax.ShapeDtypeStruct((m, c), jnp.float32),
        scratch_types=[
            pltpu.VMEM((ch,), jnp.int32),
            pltpu.VMEM((ch, c), jnp.float32),
            pltpu.SemaphoreType.DMA,
        ],
    )
    def k(table_hbm, idx_hbm, out_hbm, idx_v, rows_v, sem):
        wid = lax.axis_index("s") * _NC + lax.axis_index("c")
        base = wid * b_per_w

        def body(t, _):
            off = base + t * ch
            pltpu.sync_copy(idx_hbm.at[pl.ds(off, ch)], idx_v)
            pltpu.async_copy(table_hbm.at[idx_v], rows_v, sem).wait()
            pltpu.sync_copy(rows_v, out_hbm.at[pl.ds(off, ch)])
            return 0

        lax.fori_loop(0, n_chunks, body, 0)

    return k(table, idx)


def _pad_lanes(x):
    c = x.shape[-1]
    cp = -(-c // 16) * 16
    if cp == c:
        return x
    pad = [(0, 0)] * (x.ndim - 1) + [(0, cp - c)]
    return jnp.pad(x, pad)


def _index_points(points, idx):
    b, n, c = points.shape
    table = _pad_lanes(points).reshape(b * n, -1)
    idx = jnp.clip(idx, 0, n - 1)
    flat = (idx + (jnp.arange(b, dtype=jnp.int32) * n).reshape(
        (b,) + (1,) * (idx.ndim - 1))).reshape(-1).astype(jnp.int32)
    rows = _sc_gather_rows(table, flat, table.shape[-1])
    out = rows[:, :c].reshape(idx.shape + (c,))
    return lax.optimization_barrier(out)


def _square_distance(src, dst):
    d = -2.0 * jnp.einsum('bsc,bnc->bsn', src, dst)
    d = d + jnp.sum(src ** 2, axis=-1)[:, :, None]
    d = d + jnp.sum(dst ** 2, axis=-1)[:, None, :]
    return d


def _fps(xyz, npoint):
    b, n, _ = xyz.shape
    def body(i, state):
        centroids, distance, farthest = state
        centroids = centroids.at[:, i].set(farthest)
        centroid = jnp.take_along_axis(
            xyz, jnp.broadcast_to(farthest[:, None, None], (b, 1, 3)), axis=1)
        dist = jnp.sum((xyz - centroid) ** 2, axis=-1)
        distance = jnp.minimum(distance, dist)
        farthest = jnp.argmax(distance, axis=-1).astype(jnp.int32)
        return centroids, distance, farthest
    init = (jnp.zeros((b, npoint), jnp.int32),
            jnp.full((b, n), 1e10, jnp.float32), jnp.zeros((b,), jnp.int32))
    centroids, _, _ = jax.lax.fori_loop(0, npoint, body, init)
    return centroids


def _ball_query(radius, nsample, xyz, new_xyz):
    b, n, _ = xyz.shape
    s = new_xyz.shape[1]
    sqrdists = _square_distance(new_xyz, xyz)
    group_idx = jnp.broadcast_to(jnp.arange(n, dtype=jnp.int32), (b, s, n))
    group_idx = jnp.where(sqrdists > radius ** 2, n, group_idx)
    group_idx = jnp.sort(group_idx, axis=-1)[:, :, :nsample]
    group_first = group_idx[:, :, :1]
    group_idx = jnp.where(group_idx == n, group_first, group_idx)
    return group_idx


def _conv_bn_relu(x, W, b, g, be):
    y = jnp.einsum('oc,bcsk->bosk', W, x) + b[None, :, None, None]
    mean = jnp.mean(y, axis=(0, 2, 3), keepdims=True)
    var = jnp.mean((y - mean) ** 2, axis=(0, 2, 3), keepdims=True)
    y = (y - mean) / jnp.sqrt(var + 1e-5)
    y = y * g[None, :, None, None] + be[None, :, None, None]
    return jax.nn.relu(y)


def _sa_module(xyz, features, npoint, radii, nsamples, scale_params):
    fps_idx = _fps(xyz, npoint)
    new_xyz = _index_points(xyz, fps_idx)
    outs = []
    for radius, nsample, layers in zip(radii, nsamples, scale_params):
        idx = _ball_query(radius, nsample, xyz, new_xyz)
        grouped_xyz = _index_points(xyz, idx) - new_xyz[:, :, None, :]
        nf = jnp.transpose(grouped_xyz, (0, 3, 1, 2))
        if features is not None:
            gf = jnp.transpose(
                _index_points(jnp.transpose(features, (0, 2, 1)), idx),
                (0, 3, 1, 2))
            nf = jnp.concatenate([nf, gf], axis=1)
        for (W, b, g, be) in layers:
            nf = _conv_bn_relu(nf, W, b, g, be)
        outs.append(jnp.max(nf, axis=-1))
    return new_xyz, jnp.concatenate(outs, axis=1)


def _fp_module(unknown, known, unknow_feats, known_feats, layers):
    d = _square_distance(unknown, known)
    neg_dist, idx = jax.lax.top_k(-d, 3)
    dist = jnp.maximum(-neg_dist, 0.0)
    dist_recip = 1.0 / (dist + 1e-8)
    norm = jnp.sum(dist_recip, axis=2, keepdims=True)
    weight = dist_recip / norm
    neighbors = _index_points(jnp.transpose(known_feats, (0, 2, 1)), idx)
    interpolated = jnp.transpose(
        jnp.sum(neighbors * weight[..., None], axis=2), (0, 2, 1))
    if unknow_feats is not None:
        x = jnp.concatenate([interpolated, unknow_feats], axis=1)
    else:
        x = interpolated
    x = x[..., None]
    for (W, b, g, be) in layers:
        x = _conv_bn_relu(x, W, b, g, be)
    return x[..., 0]


def _identity_pallas(x):
    def body(x_ref, o_ref):
        o_ref[...] = x_ref[...]
    b = x.shape[0]
    blk = (1,) + x.shape[1:]
    idx = lambda i: (i,) + (0,) * (len(x.shape) - 1)
    return pl.pallas_call(
        body,
        grid=(b,),
        in_specs=[pl.BlockSpec(blk, idx)],
        out_specs=pl.BlockSpec(blk, idx),
        out_shape=jax.ShapeDtypeStruct(x.shape, x.dtype))(x)


def kernel(pointcloud, params):
    xyz = pointcloud[..., 0:3]
    features = jnp.transpose(pointcloud[..., 3:], (0, 2, 1))
    l_xyz = [xyz]
    l_features = [features]
    for i in range(4):
        nx, nf = _sa_module(l_xyz[i], l_features[i], _NPOINTS[i], _RADII[i],
                            _NSAMPLES, params['sa'][i])
        l_xyz.append(nx)
        l_features.append(nf)
    for i in range(-1, -5, -1):
        l_features[i - 1] = _fp_module(
            l_xyz[i - 1], l_xyz[i], l_features[i - 1], l_features[i],
            params['fp'][i])
    return _identity_pallas(l_features[0])
